# Initial kernel scaffold; baseline (speedup 1.0000x reference)
import jax
import jax.numpy as jnp
from jax.experimental import pallas as pl


def kernel(x, edge_index, W1, b1, W2, b2, Wfc, bfc):
    raise NotImplementedError("write your pallas kernel here")


# speedup vs baseline: 33.2693x; 33.2693x over previous
import functools

import jax
import jax.numpy as jnp
from jax import lax
from jax.experimental import pallas as pl
from jax.experimental.pallas import tpu as pltpu
from jax.experimental.pallas import tpu_sc as plsc

_N = 10000
_E = 320000
_D = 128
_H = 16
_O = 2

_NC = 2
_NS = 16
_NW = _NC * _NS
_C = 128
_K = 79
_EPAD = _NW * _K * _C
_RT = 640
_NPAD = _NS * _RT

_sc_mesh = plsc.VectorSubcoreMesh(core_axis_name="c", subcore_axis_name="s")


def _deg_body(dstr_hbm, out_hbm, dst_v, ones_v, stage_v, deg_sh, gsem):
    c = lax.axis_index("c")
    s = lax.axis_index("s")
    wid = c * _NS + s

    for i in range(8):
        ones_v[pl.ds(16 * i, 16)] = jnp.ones((16,), jnp.float32)

    def _zero(i, carry):
        stage_v[pl.ds(i * 16, 16)] = jnp.zeros((16,), jnp.float32)
        return carry

    lax.fori_loop(0, _RT // 16, _zero, 0)

    pltpu.sync_copy(dstr_hbm.at[wid], dst_v)
    pltpu.sync_copy(stage_v, deg_sh.at[pl.ds(s * _RT, _RT)])
    plsc.subcore_barrier()

    def _chunk(j, carry):
        pltpu.sync_copy(ones_v, deg_sh.at[dst_v.at[j]], add=True)
        return carry

    lax.fori_loop(0, _K, _chunk, 0)
    plsc.subcore_barrier()
    pltpu.sync_copy(deg_sh.at[pl.ds(s * _RT, _RT)], stage_v)
    pltpu.sync_copy(stage_v, out_hbm.at[c, pl.ds(s * _RT, _RT)])


_deg_call = pl.kernel(
    _deg_body,
    out_type=jax.ShapeDtypeStruct((_NC, _NPAD), jnp.float32),
    mesh=_sc_mesh,
    scratch_types=[
        pltpu.VMEM((_K, _C), jnp.int32),
        pltpu.VMEM((_C,), jnp.float32),
        pltpu.VMEM((_RT,), jnp.float32),
        pltpu.VMEM_SHARED((_NPAD,), jnp.float32),
        pltpu.SemaphoreType.DMA,
    ],
)


def _msg_body(y_hbm, srcr_hbm, dstr_hbm, out_hbm, src_v, dst_v, rows_v, stage_v,
              acc_sh, gsem):
    c = lax.axis_index("c")
    s = lax.axis_index("s")
    wid = c * _NS + s

    def _zero(i, carry):
        stage_v[i] = jnp.zeros((16,), jnp.float32)
        return carry

    lax.fori_loop(0, _RT, _zero, 0)

    pltpu.sync_copy(srcr_hbm.at[wid], src_v)
    pltpu.sync_copy(dstr_hbm.at[wid], dst_v)
    pltpu.sync_copy(stage_v, acc_sh.at[pl.ds(s * _RT, _RT)])
    plsc.subcore_barrier()

    def _chunk(j, carry):
        pltpu.async_copy(y_hbm.at[src_v.at[j]], rows_v, gsem).wait()
        pltpu.sync_copy(rows_v, acc_sh.at[dst_v.at[j]], add=True)
        return carry

    lax.fori_loop(0, _K, _chunk, 0)
    plsc.subcore_barrier()
    pltpu.sync_copy(acc_sh.at[pl.ds(s * _RT, _RT)], stage_v)
    pltpu.sync_copy(stage_v, out_hbm.at[c, pl.ds(s * _RT, _RT)])


_msg_call = pl.kernel(
    _msg_body,
    out_type=jax.ShapeDtypeStruct((_NC, _NPAD, _H), jnp.float32),
    mesh=_sc_mesh,
    scratch_types=[
        pltpu.VMEM((_K, _C), jnp.int32),
        pltpu.VMEM((_K, _C), jnp.int32),
        pltpu.VMEM((_C, _H), jnp.float32),
        pltpu.VMEM((_RT, _H), jnp.float32),
        pltpu.VMEM_SHARED((_NPAD, _H), jnp.float32),
        pltpu.SemaphoreType.DMA,
    ],
    compiler_params=pltpu.CompilerParams(use_tc_tiling_on_sc=False),
)


def _mm1_body(x_ref, w_ref, o_ref):
    o_ref[...] = jnp.dot(x_ref[...], w_ref[...],
                         preferred_element_type=jnp.float32)


def _scale_body(xw_ref, degp_ref, y_ref, inv_ref):
    deg = 1.0 + degp_ref[0, :_N, :] + degp_ref[1, :_N, :]
    inv = lax.rsqrt(deg)
    inv_ref[...] = inv
    y_ref[...] = xw_ref[...] * inv


def _mid_body(p_ref, y_ref, inv_ref, b_ref, w_ref, y2_ref):
    ssum = p_ref[0, :_N, :] + p_ref[1, :_N, :] + y_ref[...]
    inv = inv_ref[...]
    h = jnp.maximum(ssum * inv + b_ref[...], 0.0)
    y2_ref[...] = jnp.dot(h, w_ref[...],
                          preferred_element_type=jnp.float32) * inv


def _fin_body(p_ref, y2_ref, inv_ref, b_ref, wfc_ref, bfc_ref, o_ref):
    ssum = p_ref[0, :_N, :] + p_ref[1, :_N, :] + y2_ref[...]
    h = jnp.maximum(ssum * inv_ref[...] + b_ref[...], 0.0)
    logits = jnp.dot(h, wfc_ref[...],
                     preferred_element_type=jnp.float32) + bfc_ref[...]
    m = jnp.max(logits, axis=1, keepdims=True)
    lse = m + jnp.log(jnp.sum(jnp.exp(logits - m), axis=1, keepdims=True))
    o_ref[...] = logits - lse


def _tc_call(body, *out_shapes):
    return pl.pallas_call(
        body,
        out_shape=(tuple(out_shapes) if len(out_shapes) > 1 else out_shapes[0]),
    )


def kernel(x, edge_index, W1, b1, W2, b2, Wfc, bfc):
    src = edge_index[0]
    dst = edge_index[1]
    pad = _EPAD - _E
    srcr = jnp.concatenate([src, jnp.zeros((pad,), src.dtype)]).reshape(_NW, _K, _C)
    dstr = jnp.concatenate([dst, jnp.full((pad,), _N, dst.dtype)]).reshape(_NW, _K, _C)

    degp = _deg_call(dstr).reshape(_NC, _NPAD, 1)
    xw1 = _tc_call(_mm1_body, jax.ShapeDtypeStruct((_N, _H), jnp.float32))(x, W1)
    y1, inv = _tc_call(
        _scale_body,
        jax.ShapeDtypeStruct((_N, _H), jnp.float32),
        jax.ShapeDtypeStruct((_N, 1), jnp.float32),
    )(xw1, degp)

    p1 = _msg_call(y1, srcr, dstr)
    y2 = _tc_call(_mid_body, jax.ShapeDtypeStruct((_N, _H), jnp.float32))(
        p1, y1, inv, b1.reshape(1, _H), W2)

    p2 = _msg_call(y2, srcr, dstr)
    return _tc_call(_fin_body, jax.ShapeDtypeStruct((_N, _O), jnp.float32))(
        p2, y2, inv, b2.reshape(1, _H), Wfc, bfc.reshape(1, _O))


# speedup vs baseline: 34.9786x; 1.0514x over previous
import functools

import jax
import jax.numpy as jnp
from jax import lax
from jax.experimental import pallas as pl
from jax.experimental.pallas import tpu as pltpu
from jax.experimental.pallas import tpu_sc as plsc

_N = 10000
_E = 320000
_D = 128
_H = 16
_O = 2

_NC = 2
_NS = 16
_NW = _NC * _NS
_C = 128
_K = 80
_EPAD = _NW * _K * _C
_RT = 640
_NPAD = _NS * _RT
_NB = 4

_sc_mesh = plsc.VectorSubcoreMesh(core_axis_name="c", subcore_axis_name="s")


def _deg_body(dstr_hbm, out_hbm, dst_v, ones_v, stage_v, deg_sh, gsem):
    c = lax.axis_index("c")
    s = lax.axis_index("s")
    wid = c * _NS + s

    for i in range(8):
        ones_v[pl.ds(16 * i, 16)] = jnp.ones((16,), jnp.float32)

    def _zero(i, carry):
        stage_v[pl.ds(i * 16, 16)] = jnp.zeros((16,), jnp.float32)
        return carry

    lax.fori_loop(0, _RT // 16, _zero, 0)

    pltpu.sync_copy(dstr_hbm.at[wid], dst_v)
    pltpu.sync_copy(stage_v, deg_sh.at[pl.ds(s * _RT, _RT)])
    plsc.subcore_barrier()

    def _group(g, carry):
        cps = [pltpu.async_copy(ones_v, deg_sh.at[dst_v.at[g * 16 + b]], gsem,
                                add=True)
               for b in range(16)]
        for cp in cps:
            cp.wait()
        return carry

    lax.fori_loop(0, _K // 16, _group, 0)
    plsc.subcore_barrier()
    pltpu.sync_copy(deg_sh.at[pl.ds(s * _RT, _RT)], stage_v)
    pltpu.sync_copy(stage_v, out_hbm.at[c, pl.ds(s * _RT, _RT)])


_deg_call = pl.kernel(
    _deg_body,
    out_type=jax.ShapeDtypeStruct((_NC, _NPAD), jnp.float32),
    mesh=_sc_mesh,
    scratch_types=[
        pltpu.VMEM((_K, _C), jnp.int32),
        pltpu.VMEM((_C,), jnp.float32),
        pltpu.VMEM((_RT,), jnp.float32),
        pltpu.VMEM_SHARED((_NPAD,), jnp.float32),
        pltpu.SemaphoreType.DMA,
    ],
)


def _msg_body(y_hbm, srcr_hbm, dstr_hbm, out_hbm, src_v, dst_v, rows_v, stage_v,
              acc_sh, gsems, ssems):
    c = lax.axis_index("c")
    s = lax.axis_index("s")
    wid = c * _NS + s

    def _zero(i, carry):
        stage_v[i] = jnp.zeros((16,), jnp.float32)
        return carry

    lax.fori_loop(0, _RT, _zero, 0)

    pltpu.sync_copy(srcr_hbm.at[wid], src_v)
    pltpu.sync_copy(dstr_hbm.at[wid], dst_v)
    pltpu.sync_copy(stage_v, acc_sh.at[pl.ds(s * _RT, _RT)])
    plsc.subcore_barrier()

    def _gather(j, b):
        return pltpu.async_copy(y_hbm.at[src_v.at[j]], rows_v.at[b], gsems[b])

    def _scatter(j, b):
        return pltpu.async_copy(rows_v.at[b], acc_sh.at[dst_v.at[j]], ssems[b],
                                add=True)

    for b in range(_NB):
        _gather(b, b)

    def _quad(q, carry):
        j0 = q * _NB
        for b in range(_NB):
            pltpu.make_async_copy(y_hbm.at[src_v.at[j0 + b]], rows_v.at[b],
                                  gsems[b]).wait()
            _scatter(j0 + b, b)
        for b in range(_NB):
            pltpu.make_async_copy(rows_v.at[b], acc_sh.at[dst_v.at[j0 + b]],
                                  ssems[b]).wait()
            _gather(j0 + _NB + b, b)
        return carry

    lax.fori_loop(0, _K // _NB - 1, _quad, 0)

    jlast = _K - _NB
    for b in range(_NB):
        pltpu.make_async_copy(y_hbm.at[src_v.at[jlast + b]], rows_v.at[b],
                              gsems[b]).wait()
        _scatter(jlast + b, b)
    for b in range(_NB):
        pltpu.make_async_copy(rows_v.at[b], acc_sh.at[dst_v.at[jlast + b]],
                              ssems[b]).wait()

    plsc.subcore_barrier()
    pltpu.sync_copy(acc_sh.at[pl.ds(s * _RT, _RT)], stage_v)
    pltpu.sync_copy(stage_v, out_hbm.at[c, pl.ds(s * _RT, _RT)])


_msg_call = pl.kernel(
    _msg_body,
    out_type=jax.ShapeDtypeStruct((_NC, _NPAD, _H), jnp.float32),
    mesh=_sc_mesh,
    scratch_types=[
        pltpu.VMEM((_K, _C), jnp.int32),
        pltpu.VMEM((_K, _C), jnp.int32),
        pltpu.VMEM((_NB, _C, _H), jnp.float32),
        pltpu.VMEM((_RT, _H), jnp.float32),
        pltpu.VMEM_SHARED((_NPAD, _H), jnp.float32),
        [pltpu.SemaphoreType.DMA] * _NB,
        [pltpu.SemaphoreType.DMA] * _NB,
    ],
    compiler_params=pltpu.CompilerParams(use_tc_tiling_on_sc=False),
)


def _scale_body(x_ref, w_ref, degp_ref, y_ref, inv_ref):
    deg = 1.0 + degp_ref[0, :_N, :] + degp_ref[1, :_N, :]
    inv = lax.rsqrt(deg)
    inv_ref[...] = inv
    xw = jnp.dot(x_ref[...], w_ref[...], preferred_element_type=jnp.float32)
    y_ref[...] = xw * inv


def _mid_body(p_ref, y_ref, inv_ref, b_ref, w_ref, y2_ref):
    ssum = p_ref[0, :_N, :] + p_ref[1, :_N, :] + y_ref[...]
    inv = inv_ref[...]
    h = jnp.maximum(ssum * inv + b_ref[...], 0.0)
    y2_ref[...] = jnp.dot(h, w_ref[...],
                          preferred_element_type=jnp.float32) * inv


def _fin_body(p_ref, y2_ref, inv_ref, b_ref, wfc_ref, bfc_ref, o_ref):
    ssum = p_ref[0, :_N, :] + p_ref[1, :_N, :] + y2_ref[...]
    h = jnp.maximum(ssum * inv_ref[...] + b_ref[...], 0.0)
    logits = jnp.dot(h, wfc_ref[...],
                     preferred_element_type=jnp.float32) + bfc_ref[...]
    m = jnp.max(logits, axis=1, keepdims=True)
    lse = m + jnp.log(jnp.sum(jnp.exp(logits - m), axis=1, keepdims=True))
    o_ref[...] = logits - lse


def _tc_call(body, *out_shapes):
    return pl.pallas_call(
        body,
        out_shape=(tuple(out_shapes) if len(out_shapes) > 1 else out_shapes[0]),
    )


def kernel(x, edge_index, W1, b1, W2, b2, Wfc, bfc):
    src = edge_index[0]
    dst = edge_index[1]
    pad = _EPAD - _E
    srcr = jnp.concatenate([src, jnp.zeros((pad,), src.dtype)]).reshape(_NW, _K, _C)
    dstr = jnp.concatenate([dst, jnp.full((pad,), _N, dst.dtype)]).reshape(_NW, _K, _C)

    degp = _deg_call(dstr).reshape(_NC, _NPAD, 1)
    y1, inv = _tc_call(
        _scale_body,
        jax.ShapeDtypeStruct((_N, _H), jnp.float32),
        jax.ShapeDtypeStruct((_N, 1), jnp.float32),
    )(x, W1, degp)

    p1 = _msg_call(y1, srcr, dstr)
    y2 = _tc_call(_mid_body, jax.ShapeDtypeStruct((_N, _H), jnp.float32))(
        p1, y1, inv, b1.reshape(1, _H), W2)

    p2 = _msg_call(y2, srcr, dstr)
    return _tc_call(_fin_body, jax.ShapeDtypeStruct((_N, _O), jnp.float32))(
        p2, y2, inv, b2.reshape(1, _H), Wfc, bfc.reshape(1, _O))


# speedup vs baseline: 38.4854x; 1.1003x over previous
import functools

import jax
import jax.numpy as jnp
from jax import lax
from jax.experimental import pallas as pl
from jax.experimental.pallas import tpu as pltpu
from jax.experimental.pallas import tpu_sc as plsc

_N = 10000
_E = 320000
_D = 128
_H = 16
_O = 2

_NC = 2
_NS = 16
_NW = _NC * _NS
_C = 128
_K = 80
_EPAD = _NW * _K * _C
_RT = 640
_NPAD = _NS * _RT
_NB = 4

_sc_mesh = plsc.VectorSubcoreMesh(core_axis_name="c", subcore_axis_name="s")


def _deg_body(dstr_hbm, out_hbm, dst_v, ones_v, stage_v, deg_sh, gsem):
    c = lax.axis_index("c")
    s = lax.axis_index("s")
    wid = c * _NS + s

    for i in range(8):
        ones_v[pl.ds(16 * i, 16)] = jnp.ones((16,), jnp.float32)

    def _zero(i, carry):
        stage_v[pl.ds(i * 16, 16)] = jnp.zeros((16,), jnp.float32)
        return carry

    lax.fori_loop(0, _RT // 16, _zero, 0)

    pltpu.sync_copy(dstr_hbm.at[wid], dst_v)
    pltpu.sync_copy(stage_v, deg_sh.at[pl.ds(s * _RT, _RT)])
    plsc.subcore_barrier()

    def _group(g, carry):
        cps = [pltpu.async_copy(ones_v, deg_sh.at[dst_v.at[g * 16 + b]], gsem,
                                add=True)
               for b in range(16)]
        for cp in cps:
            cp.wait()
        return carry

    lax.fori_loop(0, _K // 16, _group, 0)
    plsc.subcore_barrier()
    pltpu.sync_copy(deg_sh.at[pl.ds(s * _RT, _RT)], stage_v)
    pltpu.sync_copy(stage_v, out_hbm.at[c, pl.ds(s * _RT, _RT)])


_deg_call = pl.kernel(
    _deg_body,
    out_type=jax.ShapeDtypeStruct((_NC, _NPAD), jnp.float32),
    mesh=_sc_mesh,
    scratch_types=[
        pltpu.VMEM((_K, _C), jnp.int32),
        pltpu.VMEM((_C,), jnp.float32),
        pltpu.VMEM((_RT,), jnp.float32),
        pltpu.VMEM_SHARED((_NPAD,), jnp.float32),
        pltpu.SemaphoreType.DMA,
    ],
    compiler_params=pltpu.CompilerParams(use_tc_tiling_on_sc=False),
)


def _msg_body(y_hbm, srcr_hbm, dstr_hbm, out_hbm, src_v, dst_v, rows_v, stage_v,
              acc_sh, gsems, ssems):
    c = lax.axis_index("c")
    s = lax.axis_index("s")
    wid = c * _NS + s

    def _zero(i, carry):
        stage_v[i] = jnp.zeros((16,), jnp.float32)
        return carry

    lax.fori_loop(0, _RT, _zero, 0)

    pltpu.sync_copy(srcr_hbm.at[wid], src_v)
    pltpu.sync_copy(dstr_hbm.at[wid], dst_v)
    pltpu.sync_copy(stage_v, acc_sh.at[pl.ds(s * _RT, _RT)])
    plsc.subcore_barrier()

    def _gather(j, b):
        return pltpu.async_copy(y_hbm.at[src_v.at[j]], rows_v.at[b], gsems[b])

    def _scatter(j, b):
        return pltpu.async_copy(rows_v.at[b], acc_sh.at[dst_v.at[j]], ssems[b],
                                add=True)

    for b in range(_NB):
        _gather(b, b)

    def _quad(q, carry):
        j0 = q * _NB
        for b in range(_NB):
            pltpu.make_async_copy(y_hbm.at[src_v.at[j0 + b]], rows_v.at[b],
                                  gsems[b]).wait()
            _scatter(j0 + b, b)
        for b in range(_NB):
            pltpu.make_async_copy(rows_v.at[b], acc_sh.at[dst_v.at[j0 + b]],
                                  ssems[b]).wait()
            _gather(j0 + _NB + b, b)
        return carry

    lax.fori_loop(0, _K // _NB - 1, _quad, 0)

    jlast = _K - _NB
    for b in range(_NB):
        pltpu.make_async_copy(y_hbm.at[src_v.at[jlast + b]], rows_v.at[b],
                              gsems[b]).wait()
        _scatter(jlast + b, b)
    for b in range(_NB):
        pltpu.make_async_copy(rows_v.at[b], acc_sh.at[dst_v.at[jlast + b]],
                              ssems[b]).wait()

    plsc.subcore_barrier()
    pltpu.sync_copy(acc_sh.at[pl.ds(s * _RT, _RT)], stage_v)
    pltpu.sync_copy(stage_v, out_hbm.at[c, pl.ds(s * _RT, _RT)])


_msg_call = pl.kernel(
    _msg_body,
    out_type=jax.ShapeDtypeStruct((_NC, _NPAD, _H), jnp.float32),
    mesh=_sc_mesh,
    scratch_types=[
        pltpu.VMEM((_K, _C), jnp.int32),
        pltpu.VMEM((_K, _C), jnp.int32),
        pltpu.VMEM((_NB, _C, _H), jnp.float32),
        pltpu.VMEM((_RT, _H), jnp.float32),
        pltpu.VMEM_SHARED((_NPAD, _H), jnp.float32),
        [pltpu.SemaphoreType.DMA] * _NB,
        [pltpu.SemaphoreType.DMA] * _NB,
    ],
    compiler_params=pltpu.CompilerParams(use_tc_tiling_on_sc=False),
)


def _scale_body(x_ref, w_ref, degp_ref, y_ref, inv_ref):
    deg_row = 1.0 + degp_ref[0:1, :_N] + degp_ref[1:2, :_N]
    inv_col = jnp.transpose(lax.rsqrt(deg_row))
    inv = inv_col * jnp.ones((1, _H), jnp.float32)
    inv_ref[...] = inv
    xw = jnp.dot(x_ref[...], w_ref[...], preferred_element_type=jnp.float32)
    y_ref[...] = xw * inv


def _mid_body(p_ref, y_ref, inv_ref, b_ref, w_ref, y2_ref):
    ssum = p_ref[0, :_N, :] + p_ref[1, :_N, :] + y_ref[...]
    inv = inv_ref[...]
    h = jnp.maximum(ssum * inv + b_ref[...], 0.0)
    y2_ref[...] = jnp.dot(h, w_ref[...],
                          preferred_element_type=jnp.float32) * inv


def _fin_body(p_ref, y2_ref, inv_ref, b_ref, wfc_ref, bfc_ref, o_ref):
    ssum = p_ref[0, :_N, :] + p_ref[1, :_N, :] + y2_ref[...]
    h = jnp.maximum(ssum * inv_ref[...] + b_ref[...], 0.0)
    logits = jnp.dot(h, wfc_ref[...],
                     preferred_element_type=jnp.float32) + bfc_ref[...]
    m = jnp.max(logits, axis=1, keepdims=True)
    lse = m + jnp.log(jnp.sum(jnp.exp(logits - m), axis=1, keepdims=True))
    o_ref[...] = logits - lse


def _tc_call(body, *out_shapes):
    return pl.pallas_call(
        body,
        out_shape=(tuple(out_shapes) if len(out_shapes) > 1 else out_shapes[0]),
    )


def kernel(x, edge_index, W1, b1, W2, b2, Wfc, bfc):
    src = edge_index[0]
    dst = edge_index[1]
    pad = _EPAD - _E
    srcr = jnp.concatenate([src, jnp.zeros((pad,), src.dtype)]).reshape(_NW, _K, _C)
    dstr = jnp.concatenate([dst, jnp.full((pad,), _N, dst.dtype)]).reshape(_NW, _K, _C)

    degp = _deg_call(dstr)
    y1, inv = _tc_call(
        _scale_body,
        jax.ShapeDtypeStruct((_N, _H), jnp.float32),
        jax.ShapeDtypeStruct((_N, _H), jnp.float32),
    )(x, W1, degp)

    p1 = _msg_call(y1, srcr, dstr)
    y2 = _tc_call(_mid_body, jax.ShapeDtypeStruct((_N, _H), jnp.float32))(
        p1, y1, inv, b1.reshape(1, _H), W2)

    p2 = _msg_call(y2, srcr, dstr)
    return _tc_call(_fin_body, jax.ShapeDtypeStruct((_N, _O), jnp.float32))(
        p2, y2, inv, b2.reshape(1, _H), Wfc, bfc.reshape(1, _O))


# speedup vs baseline: 40.0700x; 1.0412x over previous
import functools

import jax
import jax.numpy as jnp
from jax import lax
from jax.experimental import pallas as pl
from jax.experimental.pallas import tpu as pltpu
from jax.experimental.pallas import tpu_sc as plsc

_N = 10000
_E = 320000
_D = 128
_H = 16
_O = 2

_NC = 2
_NS = 16
_NW = _NC * _NS
_C = 128
_K0 = 96
_K1 = 64
_KMAX = max(_K0, _K1)
_NCH = _NS * (_K0 + _K1)
_EPAD = _NCH * _C
_RT = 640
_NPAD = _NS * _RT
_NB = 4

_sc_mesh = plsc.VectorSubcoreMesh(core_axis_name="c", subcore_axis_name="s")


def _deg_body(dstr_hbm, out_hbm, dst_v, ones_v, stage_v, deg_sh, gsem):
    c = lax.axis_index("c")
    s = lax.axis_index("s")
    nk = jnp.where(c == 0, _K0, _K1)
    base = jnp.where(c == 0, s * _K0, _NS * _K0 + s * _K1)

    for i in range(8):
        ones_v[pl.ds(16 * i, 16)] = jnp.ones((16,), jnp.float32)

    def _zero(i, carry):
        stage_v[pl.ds(i * 16, 16)] = jnp.zeros((16,), jnp.float32)
        return carry

    lax.fori_loop(0, _RT // 16, _zero, 0)

    pltpu.sync_copy(dstr_hbm.at[pl.ds(base, _KMAX)], dst_v)
    pltpu.sync_copy(stage_v, deg_sh.at[pl.ds(s * _RT, _RT)])
    plsc.subcore_barrier()

    def _group(g, carry):
        cps = [pltpu.async_copy(ones_v, deg_sh.at[dst_v.at[g * 16 + b]], gsem,
                                add=True)
               for b in range(16)]
        for cp in cps:
            cp.wait()
        return carry

    lax.fori_loop(0, nk // 16, _group, 0)
    plsc.subcore_barrier()
    pltpu.sync_copy(deg_sh.at[pl.ds(s * _RT, _RT)], stage_v)
    pltpu.sync_copy(stage_v, out_hbm.at[c, pl.ds(s * _RT, _RT)])


_deg_call = pl.kernel(
    _deg_body,
    out_type=jax.ShapeDtypeStruct((_NC, _NPAD), jnp.float32),
    mesh=_sc_mesh,
    scratch_types=[
        pltpu.VMEM((_KMAX, _C), jnp.int32),
        pltpu.VMEM((_C,), jnp.float32),
        pltpu.VMEM((_RT,), jnp.float32),
        pltpu.VMEM_SHARED((_NPAD,), jnp.float32),
        pltpu.SemaphoreType.DMA,
    ],
    compiler_params=pltpu.CompilerParams(use_tc_tiling_on_sc=False),
)


def _msg_body(y_hbm, srcr_hbm, dstr_hbm, out_hbm, src_v, dst_v, rows_v, stage_v,
              acc_sh, gsems, ssems):
    c = lax.axis_index("c")
    s = lax.axis_index("s")
    nk = jnp.where(c == 0, _K0, _K1)
    base = jnp.where(c == 0, s * _K0, _NS * _K0 + s * _K1)

    def _zero(i, carry):
        stage_v[i] = jnp.zeros((16,), jnp.float32)
        return carry

    lax.fori_loop(0, _RT, _zero, 0)

    pltpu.sync_copy(srcr_hbm.at[pl.ds(base, _KMAX)], src_v)
    pltpu.sync_copy(dstr_hbm.at[pl.ds(base, _KMAX)], dst_v)
    pltpu.sync_copy(stage_v, acc_sh.at[pl.ds(s * _RT, _RT)])
    plsc.subcore_barrier()

    def _gather(j, b):
        return pltpu.async_copy(y_hbm.at[src_v.at[j]], rows_v.at[b], gsems[b])

    def _scatter(j, b):
        return pltpu.async_copy(rows_v.at[b], acc_sh.at[dst_v.at[j]], ssems[b],
                                add=True)

    for b in range(_NB):
        _gather(b, b)

    def _quad(q, carry):
        j0 = q * _NB
        for b in range(_NB):
            pltpu.make_async_copy(y_hbm.at[src_v.at[j0 + b]], rows_v.at[b],
                                  gsems[b]).wait()
            _scatter(j0 + b, b)
        for b in range(_NB):
            pltpu.make_async_copy(rows_v.at[b], acc_sh.at[dst_v.at[j0 + b]],
                                  ssems[b]).wait()
            _gather(j0 + _NB + b, b)
        return carry

    lax.fori_loop(0, nk // _NB - 1, _quad, 0)

    jlast = nk - _NB
    for b in range(_NB):
        pltpu.make_async_copy(y_hbm.at[src_v.at[jlast + b]], rows_v.at[b],
                              gsems[b]).wait()
        _scatter(jlast + b, b)
    for b in range(_NB):
        pltpu.make_async_copy(rows_v.at[b], acc_sh.at[dst_v.at[jlast + b]],
                              ssems[b]).wait()

    plsc.subcore_barrier()
    pltpu.sync_copy(acc_sh.at[pl.ds(s * _RT, _RT)], stage_v)
    pltpu.sync_copy(stage_v, out_hbm.at[c, pl.ds(s * _RT, _RT)])


_msg_call = pl.kernel(
    _msg_body,
    out_type=jax.ShapeDtypeStruct((_NC, _NPAD, _H), jnp.float32),
    mesh=_sc_mesh,
    scratch_types=[
        pltpu.VMEM((_KMAX, _C), jnp.int32),
        pltpu.VMEM((_KMAX, _C), jnp.int32),
        pltpu.VMEM((_NB, _C, _H), jnp.float32),
        pltpu.VMEM((_RT, _H), jnp.float32),
        pltpu.VMEM_SHARED((_NPAD, _H), jnp.float32),
        [pltpu.SemaphoreType.DMA] * _NB,
        [pltpu.SemaphoreType.DMA] * _NB,
    ],
    compiler_params=pltpu.CompilerParams(use_tc_tiling_on_sc=False),
)


def _scale_body(x_ref, w_ref, degp_ref, y_ref, inv_ref):
    deg_row = 1.0 + degp_ref[0:1, :_N] + degp_ref[1:2, :_N]
    inv_col = jnp.transpose(lax.rsqrt(deg_row))
    inv = inv_col * jnp.ones((1, _H), jnp.float32)
    inv_ref[...] = inv
    xw = jnp.dot(x_ref[...], w_ref[...], preferred_element_type=jnp.float32)
    y_ref[...] = xw * inv


def _mid_body(p_ref, y_ref, inv_ref, b_ref, w_ref, y2_ref):
    ssum = p_ref[0, :_N, :] + p_ref[1, :_N, :] + y_ref[...]
    inv = inv_ref[...]
    h = jnp.maximum(ssum * inv + b_ref[...], 0.0)
    y2_ref[...] = jnp.dot(h, w_ref[...],
                          preferred_element_type=jnp.float32) * inv


def _fin_body(p_ref, y2_ref, inv_ref, b_ref, wfc_ref, bfc_ref, o_ref):
    ssum = p_ref[0, :_N, :] + p_ref[1, :_N, :] + y2_ref[...]
    h = jnp.maximum(ssum * inv_ref[...] + b_ref[...], 0.0)
    logits = jnp.dot(h, wfc_ref[...],
                     preferred_element_type=jnp.float32) + bfc_ref[...]
    m = jnp.max(logits, axis=1, keepdims=True)
    lse = m + jnp.log(jnp.sum(jnp.exp(logits - m), axis=1, keepdims=True))
    o_ref[...] = logits - lse


def _tc_call(body, *out_shapes):
    return pl.pallas_call(
        body,
        out_shape=(tuple(out_shapes) if len(out_shapes) > 1 else out_shapes[0]),
    )


def kernel(x, edge_index, W1, b1, W2, b2, Wfc, bfc):
    src = edge_index[0]
    dst = edge_index[1]
    pad = _EPAD - _E + _KMAX * _C
    srcr = jnp.concatenate([src, jnp.zeros((pad,), src.dtype)]).reshape(-1, _C)
    dstr = jnp.concatenate([dst, jnp.full((pad,), _N, dst.dtype)]).reshape(-1, _C)

    degp = _deg_call(dstr)
    y1, inv = _tc_call(
        _scale_body,
        jax.ShapeDtypeStruct((_N, _H), jnp.float32),
        jax.ShapeDtypeStruct((_N, _H), jnp.float32),
    )(x, W1, degp)

    p1 = _msg_call(y1, srcr, dstr)
    y2 = _tc_call(_mid_body, jax.ShapeDtypeStruct((_N, _H), jnp.float32))(
        p1, y1, inv, b1.reshape(1, _H), W2)

    p2 = _msg_call(y2, srcr, dstr)
    return _tc_call(_fin_body, jax.ShapeDtypeStruct((_N, _O), jnp.float32))(
        p2, y2, inv, b2.reshape(1, _H), Wfc, bfc.reshape(1, _O))


# speedup vs baseline: 40.5691x; 1.0125x over previous
import functools

import jax
import jax.numpy as jnp
from jax import lax
from jax.experimental import pallas as pl
from jax.experimental.pallas import tpu as pltpu
from jax.experimental.pallas import tpu_sc as plsc

_N = 10000
_E = 320000
_D = 128
_H = 16
_O = 2

_NC = 2
_NS = 16
_NW = _NC * _NS
_C = 128
_K0 = 112
_K1 = 48
_KMAX = max(_K0, _K1)
_NCH = _NS * (_K0 + _K1)
_EPAD = _NCH * _C
_RT = 640
_NPAD = _NS * _RT
_NB = 4

_sc_mesh = plsc.VectorSubcoreMesh(core_axis_name="c", subcore_axis_name="s")


def _deg_body(dstr_hbm, out_hbm, dst_v, ones_v, stage_v, deg_sh, gsem):
    c = lax.axis_index("c")
    s = lax.axis_index("s")
    nk = jnp.where(c == 0, _K0, _K1)
    base = jnp.where(c == 0, s * _K0, _NS * _K0 + s * _K1)

    for i in range(8):
        ones_v[pl.ds(16 * i, 16)] = jnp.ones((16,), jnp.float32)

    def _zero(i, carry):
        stage_v[pl.ds(i * 16, 16)] = jnp.zeros((16,), jnp.float32)
        return carry

    lax.fori_loop(0, _RT // 16, _zero, 0)

    pltpu.sync_copy(dstr_hbm.at[pl.ds(base, _KMAX)], dst_v)
    pltpu.sync_copy(stage_v, deg_sh.at[pl.ds(s * _RT, _RT)])
    plsc.subcore_barrier()

    def _group(g, carry):
        cps = [pltpu.async_copy(ones_v, deg_sh.at[dst_v.at[g * 16 + b]], gsem,
                                add=True)
               for b in range(16)]
        for cp in cps:
            cp.wait()
        return carry

    lax.fori_loop(0, nk // 16, _group, 0)
    plsc.subcore_barrier()
    pltpu.sync_copy(deg_sh.at[pl.ds(s * _RT, _RT)], stage_v)
    pltpu.sync_copy(stage_v, out_hbm.at[c, pl.ds(s * _RT, _RT)])


_deg_call = pl.kernel(
    _deg_body,
    out_type=jax.ShapeDtypeStruct((_NC, _NPAD), jnp.float32),
    mesh=_sc_mesh,
    scratch_types=[
        pltpu.VMEM((_KMAX, _C), jnp.int32),
        pltpu.VMEM((_C,), jnp.float32),
        pltpu.VMEM((_RT,), jnp.float32),
        pltpu.VMEM_SHARED((_NPAD,), jnp.float32),
        pltpu.SemaphoreType.DMA,
    ],
    compiler_params=pltpu.CompilerParams(use_tc_tiling_on_sc=False),
)


def _msg_body(y_hbm, srcr_hbm, dstr_hbm, out_hbm, src_v, dst_v, rows_v, stage_v,
              acc_sh, gsems, ssems):
    c = lax.axis_index("c")
    s = lax.axis_index("s")
    nk = jnp.where(c == 0, _K0, _K1)
    base = jnp.where(c == 0, s * _K0, _NS * _K0 + s * _K1)

    def _zero(i, carry):
        stage_v[i] = jnp.zeros((16,), jnp.float32)
        return carry

    lax.fori_loop(0, _RT, _zero, 0)

    pltpu.sync_copy(srcr_hbm.at[pl.ds(base, _KMAX)], src_v)
    pltpu.sync_copy(dstr_hbm.at[pl.ds(base, _KMAX)], dst_v)
    pltpu.sync_copy(stage_v, acc_sh.at[pl.ds(s * _RT, _RT)])
    plsc.subcore_barrier()

    def _gather(j, b):
        return pltpu.async_copy(y_hbm.at[src_v.at[j]], rows_v.at[b], gsems[b])

    def _scatter(j, b):
        return pltpu.async_copy(rows_v.at[b], acc_sh.at[dst_v.at[j]], ssems[b],
                                add=True)

    for b in range(_NB):
        _gather(b, b)

    def _quad(q, carry):
        j0 = q * _NB
        for b in range(_NB):
            pltpu.make_async_copy(y_hbm.at[src_v.at[j0 + b]], rows_v.at[b],
                                  gsems[b]).wait()
            _scatter(j0 + b, b)
        for b in range(_NB):
            pltpu.make_async_copy(rows_v.at[b], acc_sh.at[dst_v.at[j0 + b]],
                                  ssems[b]).wait()
            _gather(j0 + _NB + b, b)
        return carry

    lax.fori_loop(0, nk // _NB - 1, _quad, 0)

    jlast = nk - _NB
    for b in range(_NB):
        pltpu.make_async_copy(y_hbm.at[src_v.at[jlast + b]], rows_v.at[b],
                              gsems[b]).wait()
        _scatter(jlast + b, b)
    for b in range(_NB):
        pltpu.make_async_copy(rows_v.at[b], acc_sh.at[dst_v.at[jlast + b]],
                              ssems[b]).wait()

    plsc.subcore_barrier()
    pltpu.sync_copy(acc_sh.at[pl.ds(s * _RT, _RT)], stage_v)
    pltpu.sync_copy(stage_v, out_hbm.at[c, pl.ds(s * _RT, _RT)])


_msg_call = pl.kernel(
    _msg_body,
    out_type=jax.ShapeDtypeStruct((_NC, _NPAD, _H), jnp.float32),
    mesh=_sc_mesh,
    scratch_types=[
        pltpu.VMEM((_KMAX, _C), jnp.int32),
        pltpu.VMEM((_KMAX, _C), jnp.int32),
        pltpu.VMEM((_NB, _C, _H), jnp.float32),
        pltpu.VMEM((_RT, _H), jnp.float32),
        pltpu.VMEM_SHARED((_NPAD, _H), jnp.float32),
        [pltpu.SemaphoreType.DMA] * _NB,
        [pltpu.SemaphoreType.DMA] * _NB,
    ],
    compiler_params=pltpu.CompilerParams(use_tc_tiling_on_sc=False),
)


def _scale_body(x_ref, w_ref, degp_ref, y_ref, inv_ref):
    deg_row = 1.0 + degp_ref[0:1, :_N] + degp_ref[1:2, :_N]
    inv_col = jnp.transpose(lax.rsqrt(deg_row))
    inv = inv_col * jnp.ones((1, _H), jnp.float32)
    inv_ref[...] = inv
    xw = jnp.dot(x_ref[...], w_ref[...], preferred_element_type=jnp.float32)
    y_ref[...] = xw * inv


def _mid_body(p_ref, y_ref, inv_ref, b_ref, w_ref, y2_ref):
    ssum = p_ref[0, :_N, :] + p_ref[1, :_N, :] + y_ref[...]
    inv = inv_ref[...]
    h = jnp.maximum(ssum * inv + b_ref[...], 0.0)
    y2_ref[...] = jnp.dot(h, w_ref[...],
                          preferred_element_type=jnp.float32) * inv


def _fin_body(p_ref, y2_ref, inv_ref, b_ref, wfc_ref, bfc_ref, o_ref):
    ssum = p_ref[0, :_N, :] + p_ref[1, :_N, :] + y2_ref[...]
    h = jnp.maximum(ssum * inv_ref[...] + b_ref[...], 0.0)
    logits = jnp.dot(h, wfc_ref[...],
                     preferred_element_type=jnp.float32) + bfc_ref[...]
    m = jnp.max(logits, axis=1, keepdims=True)
    lse = m + jnp.log(jnp.sum(jnp.exp(logits - m), axis=1, keepdims=True))
    o_ref[...] = logits - lse


def _tc_call(body, *out_shapes):
    return pl.pallas_call(
        body,
        out_shape=(tuple(out_shapes) if len(out_shapes) > 1 else out_shapes[0]),
    )


def kernel(x, edge_index, W1, b1, W2, b2, Wfc, bfc):
    src = edge_index[0]
    dst = edge_index[1]
    pad = _EPAD - _E + _KMAX * _C
    srcr = jnp.concatenate([src, jnp.zeros((pad,), src.dtype)]).reshape(-1, _C)
    dstr = jnp.concatenate([dst, jnp.full((pad,), _N, dst.dtype)]).reshape(-1, _C)

    degp = _deg_call(dstr)
    y1, inv = _tc_call(
        _scale_body,
        jax.ShapeDtypeStruct((_N, _H), jnp.float32),
        jax.ShapeDtypeStruct((_N, _H), jnp.float32),
    )(x, W1, degp)

    p1 = _msg_call(y1, srcr, dstr)
    y2 = _tc_call(_mid_body, jax.ShapeDtypeStruct((_N, _H), jnp.float32))(
        p1, y1, inv, b1.reshape(1, _H), W2)

    p2 = _msg_call(y2, srcr, dstr)
    return _tc_call(_fin_body, jax.ShapeDtypeStruct((_N, _O), jnp.float32))(
        p2, y2, inv, b2.reshape(1, _H), Wfc, bfc.reshape(1, _O))


# speedup vs baseline: 55.5276x; 1.3687x over previous
import functools

import jax
import jax.numpy as jnp
from jax import lax
from jax.experimental import pallas as pl
from jax.experimental.pallas import tpu as pltpu
from jax.experimental.pallas import tpu_sc as plsc

_N = 10000
_E = 320000
_D = 128
_H = 16
_O = 2

_NC = 2
_NS = 16
_NW = _NC * _NS
_C = 128
_K0 = 96
_K1 = 60
_KMAX = _K0 + 4
_NCH = _E // _C
_RT = 640
_NPAD = _NS * _RT
_NB = 4

_sc_mesh = plsc.VectorSubcoreMesh(core_axis_name="c", subcore_axis_name="s")


def _chunk_range(c, s):
    nk = jnp.where(c == 0, _K0 + 4 * (s == 0).astype(jnp.int32), _K1)
    base0 = _K0 * s + 4 * (s > 0).astype(jnp.int32)
    base1 = _NS * _K0 + 4 + _K1 * s
    return nk, jnp.where(c == 0, base0, base1)


def _stage_idx(c, idx_hbm, base, idx_v):
    @pl.when(c == 0)
    def _():
        pltpu.sync_copy(idx_hbm.at[pl.ds(base, _KMAX)], idx_v)

    @pl.when(c != 0)
    def _():
        pltpu.sync_copy(idx_hbm.at[pl.ds(base, _K1)], idx_v.at[pl.ds(0, _K1)])


def _deg_body(dstr_hbm, out_hbm, dst_v, ones_v, stage_v, deg_sh, gsem):
    c = lax.axis_index("c")
    s = lax.axis_index("s")
    nk, base = _chunk_range(c, s)

    for i in range(8):
        ones_v[pl.ds(16 * i, 16)] = jnp.ones((16,), jnp.float32)

    def _zero(i, carry):
        stage_v[pl.ds(i * 16, 16)] = jnp.zeros((16,), jnp.float32)
        return carry

    lax.fori_loop(0, _RT // 16, _zero, 0)

    _stage_idx(c, dstr_hbm, base, dst_v)
    pltpu.sync_copy(stage_v, deg_sh.at[pl.ds(s * _RT, _RT)])
    plsc.subcore_barrier()

    def _group(g, carry):
        cps = [pltpu.async_copy(ones_v, deg_sh.at[dst_v.at[g * 4 + b]], gsem,
                                add=True)
               for b in range(4)]
        for cp in cps:
            cp.wait()
        return carry

    lax.fori_loop(0, nk // 4, _group, 0)
    plsc.subcore_barrier()
    pltpu.sync_copy(deg_sh.at[pl.ds(s * _RT, _RT)], stage_v)
    pltpu.sync_copy(stage_v, out_hbm.at[c, pl.ds(s * _RT, _RT)])


_deg_call = pl.kernel(
    _deg_body,
    out_type=jax.ShapeDtypeStruct((_NC, _NPAD), jnp.float32),
    mesh=_sc_mesh,
    scratch_types=[
        pltpu.VMEM((_KMAX, _C), jnp.int32),
        pltpu.VMEM((_C,), jnp.float32),
        pltpu.VMEM((_RT,), jnp.float32),
        pltpu.VMEM_SHARED((_NPAD,), jnp.float32),
        pltpu.SemaphoreType.DMA,
    ],
    compiler_params=pltpu.CompilerParams(use_tc_tiling_on_sc=False),
)


def _msg_body(y_hbm, srcr_hbm, dstr_hbm, out_hbm, src_v, dst_v, rows_v, stage_v,
              acc_sh, gsems, ssems):
    c = lax.axis_index("c")
    s = lax.axis_index("s")
    nk, base = _chunk_range(c, s)

    def _zero(i, carry):
        stage_v[i] = jnp.zeros((16,), jnp.float32)
        return carry

    lax.fori_loop(0, _RT, _zero, 0)

    _stage_idx(c, srcr_hbm, base, src_v)
    _stage_idx(c, dstr_hbm, base, dst_v)
    pltpu.sync_copy(stage_v, acc_sh.at[pl.ds(s * _RT, _RT)])
    plsc.subcore_barrier()

    def _gather(j, b):
        return pltpu.async_copy(y_hbm.at[src_v.at[j]], rows_v.at[b], gsems[b])

    def _scatter(j, b):
        return pltpu.async_copy(rows_v.at[b], acc_sh.at[dst_v.at[j]], ssems[b],
                                add=True)

    for b in range(_NB):
        _gather(b, b)

    def _quad(q, carry):
        j0 = q * _NB
        for b in range(_NB):
            pltpu.make_async_copy(y_hbm.at[src_v.at[j0 + b]], rows_v.at[b],
                                  gsems[b]).wait()
            _scatter(j0 + b, b)
        for b in range(_NB):
            pltpu.make_async_copy(rows_v.at[b], acc_sh.at[dst_v.at[j0 + b]],
                                  ssems[b]).wait()
            _gather(j0 + _NB + b, b)
        return carry

    lax.fori_loop(0, nk // _NB - 1, _quad, 0)

    jlast = nk - _NB
    for b in range(_NB):
        pltpu.make_async_copy(y_hbm.at[src_v.at[jlast + b]], rows_v.at[b],
                              gsems[b]).wait()
        _scatter(jlast + b, b)
    for b in range(_NB):
        pltpu.make_async_copy(rows_v.at[b], acc_sh.at[dst_v.at[jlast + b]],
                              ssems[b]).wait()

    plsc.subcore_barrier()
    pltpu.sync_copy(acc_sh.at[pl.ds(s * _RT, _RT)], stage_v)
    pltpu.sync_copy(stage_v, out_hbm.at[c, pl.ds(s * _RT, _RT)])


_msg_call = pl.kernel(
    _msg_body,
    out_type=jax.ShapeDtypeStruct((_NC, _NPAD, _H), jnp.float32),
    mesh=_sc_mesh,
    scratch_types=[
        pltpu.VMEM((_KMAX, _C), jnp.int32),
        pltpu.VMEM((_KMAX, _C), jnp.int32),
        pltpu.VMEM((_NB, _C, _H), jnp.float32),
        pltpu.VMEM((_RT, _H), jnp.float32),
        pltpu.VMEM_SHARED((_NPAD, _H), jnp.float32),
        [pltpu.SemaphoreType.DMA] * _NB,
        [pltpu.SemaphoreType.DMA] * _NB,
    ],
    compiler_params=pltpu.CompilerParams(use_tc_tiling_on_sc=False),
)


def _scale_body(x_ref, w_ref, degp_ref, y_ref, inv_ref):
    deg_row = 1.0 + degp_ref[0:1, :_N] + degp_ref[1:2, :_N]
    inv_col = jnp.transpose(lax.rsqrt(deg_row))
    inv = inv_col * jnp.ones((1, _H), jnp.float32)
    inv_ref[...] = inv
    xw = jnp.dot(x_ref[...], w_ref[...], preferred_element_type=jnp.float32)
    y_ref[...] = xw * inv


def _mid_body(p_ref, y_ref, inv_ref, b_ref, w_ref, y2_ref):
    ssum = p_ref[0, :_N, :] + p_ref[1, :_N, :] + y_ref[...]
    inv = inv_ref[...]
    h = jnp.maximum(ssum * inv + b_ref[...], 0.0)
    y2_ref[...] = jnp.dot(h, w_ref[...],
                          preferred_element_type=jnp.float32) * inv


def _fin_body(p_ref, y2_ref, inv_ref, b_ref, wfc_ref, bfc_ref, o_ref):
    ssum = p_ref[0, :_N, :] + p_ref[1, :_N, :] + y2_ref[...]
    h = jnp.maximum(ssum * inv_ref[...] + b_ref[...], 0.0)
    logits = jnp.dot(h, wfc_ref[...],
                     preferred_element_type=jnp.float32) + bfc_ref[...]
    m = jnp.max(logits, axis=1, keepdims=True)
    lse = m + jnp.log(jnp.sum(jnp.exp(logits - m), axis=1, keepdims=True))
    o_ref[...] = logits - lse


def _tc_call(body, *out_shapes):
    return pl.pallas_call(
        body,
        out_shape=(tuple(out_shapes) if len(out_shapes) > 1 else out_shapes[0]),
    )


def kernel(x, edge_index, W1, b1, W2, b2, Wfc, bfc):
    srcr = edge_index[0].reshape(_NCH, _C)
    dstr = edge_index[1].reshape(_NCH, _C)

    degp = _deg_call(dstr)
    y1, inv = _tc_call(
        _scale_body,
        jax.ShapeDtypeStruct((_N, _H), jnp.float32),
        jax.ShapeDtypeStruct((_N, _H), jnp.float32),
    )(x, W1, degp)

    p1 = _msg_call(y1, srcr, dstr)
    y2 = _tc_call(_mid_body, jax.ShapeDtypeStruct((_N, _H), jnp.float32))(
        p1, y1, inv, b1.reshape(1, _H), W2)

    p2 = _msg_call(y2, srcr, dstr)
    return _tc_call(_fin_body, jax.ShapeDtypeStruct((_N, _O), jnp.float32))(
        p2, y2, inv, b2.reshape(1, _H), Wfc, bfc.reshape(1, _O))


# speedup vs baseline: 56.8592x; 1.0240x over previous
import functools

import jax
import jax.numpy as jnp
from jax import lax
from jax.experimental import pallas as pl
from jax.experimental.pallas import tpu as pltpu
from jax.experimental.pallas import tpu_sc as plsc

_N = 10000
_E = 320000
_D = 128
_H = 16
_O = 2

_NC = 2
_NS = 16
_NW = _NC * _NS
_C = 128
_K0 = 88
_K1 = 68
_KMAX = _K0 + 4
_NCH = _E // _C
_RT = 640
_NPAD = _NS * _RT
_NB = 4

_sc_mesh = plsc.VectorSubcoreMesh(core_axis_name="c", subcore_axis_name="s")


def _chunk_range(c, s):
    nk = jnp.where(c == 0, _K0 + 4 * (s == 0).astype(jnp.int32), _K1)
    base0 = _K0 * s + 4 * (s > 0).astype(jnp.int32)
    base1 = _NS * _K0 + 4 + _K1 * s
    return nk, jnp.where(c == 0, base0, base1)


def _stage_idx(c, idx_hbm, base, idx_v):
    @pl.when(c == 0)
    def _():
        pltpu.sync_copy(idx_hbm.at[pl.ds(base, _KMAX)], idx_v)

    @pl.when(c != 0)
    def _():
        pltpu.sync_copy(idx_hbm.at[pl.ds(base, _K1)], idx_v.at[pl.ds(0, _K1)])


def _deg_body(dstr_hbm, out_hbm, dst_v, ones_v, stage_v, deg_sh, gsem):
    c = lax.axis_index("c")
    s = lax.axis_index("s")
    nk, base = _chunk_range(c, s)

    for i in range(8):
        ones_v[pl.ds(16 * i, 16)] = jnp.ones((16,), jnp.float32)

    def _zero(i, carry):
        stage_v[pl.ds(i * 16, 16)] = jnp.zeros((16,), jnp.float32)
        return carry

    lax.fori_loop(0, _RT // 16, _zero, 0)

    _stage_idx(c, dstr_hbm, base, dst_v)
    pltpu.sync_copy(stage_v, deg_sh.at[pl.ds(s * _RT, _RT)])
    plsc.subcore_barrier()

    def _group(g, carry):
        cps = [pltpu.async_copy(ones_v, deg_sh.at[dst_v.at[g * 4 + b]], gsem,
                                add=True)
               for b in range(4)]
        for cp in cps:
            cp.wait()
        return carry

    lax.fori_loop(0, nk // 4, _group, 0)
    plsc.subcore_barrier()
    pltpu.sync_copy(deg_sh.at[pl.ds(s * _RT, _RT)], stage_v)
    pltpu.sync_copy(stage_v, out_hbm.at[c, pl.ds(s * _RT, _RT)])


_deg_call = pl.kernel(
    _deg_body,
    out_type=jax.ShapeDtypeStruct((_NC, _NPAD), jnp.float32),
    mesh=_sc_mesh,
    scratch_types=[
        pltpu.VMEM((_KMAX, _C), jnp.int32),
        pltpu.VMEM((_C,), jnp.float32),
        pltpu.VMEM((_RT,), jnp.float32),
        pltpu.VMEM_SHARED((_NPAD,), jnp.float32),
        pltpu.SemaphoreType.DMA,
    ],
    compiler_params=pltpu.CompilerParams(use_tc_tiling_on_sc=False, disable_bounds_checks=True),
)


def _msg_body(y_hbm, srcr_hbm, dstr_hbm, out_hbm, src_v, dst_v, rows_v, stage_v,
              acc_sh, gsems, ssems):
    c = lax.axis_index("c")
    s = lax.axis_index("s")
    nk, base = _chunk_range(c, s)

    def _zero(i, carry):
        stage_v[i] = jnp.zeros((16,), jnp.float32)
        return carry

    lax.fori_loop(0, _RT, _zero, 0)

    _stage_idx(c, srcr_hbm, base, src_v)
    _stage_idx(c, dstr_hbm, base, dst_v)
    pltpu.sync_copy(stage_v, acc_sh.at[pl.ds(s * _RT, _RT)])
    plsc.subcore_barrier()

    def _gather(j, b):
        return pltpu.async_copy(y_hbm.at[src_v.at[j]], rows_v.at[b], gsems[b])

    def _scatter(j, b):
        return pltpu.async_copy(rows_v.at[b], acc_sh.at[dst_v.at[j]], ssems[b],
                                add=True)

    for b in range(_NB):
        _gather(b, b)

    def _quad(q, carry):
        j0 = q * _NB
        for b in range(_NB):
            pltpu.make_async_copy(y_hbm.at[src_v.at[j0 + b]], rows_v.at[b],
                                  gsems[b]).wait()
            _scatter(j0 + b, b)
        for b in range(_NB):
            pltpu.make_async_copy(rows_v.at[b], acc_sh.at[dst_v.at[j0 + b]],
                                  ssems[b]).wait()
            _gather(j0 + _NB + b, b)
        return carry

    lax.fori_loop(0, nk // _NB - 1, _quad, 0)

    jlast = nk - _NB
    for b in range(_NB):
        pltpu.make_async_copy(y_hbm.at[src_v.at[jlast + b]], rows_v.at[b],
                              gsems[b]).wait()
        _scatter(jlast + b, b)
    for b in range(_NB):
        pltpu.make_async_copy(rows_v.at[b], acc_sh.at[dst_v.at[jlast + b]],
                              ssems[b]).wait()

    plsc.subcore_barrier()
    pltpu.sync_copy(acc_sh.at[pl.ds(s * _RT, _RT)], stage_v)
    pltpu.sync_copy(stage_v, out_hbm.at[c, pl.ds(s * _RT, _RT)])


_msg_call = pl.kernel(
    _msg_body,
    out_type=jax.ShapeDtypeStruct((_NC, _NPAD, _H), jnp.float32),
    mesh=_sc_mesh,
    scratch_types=[
        pltpu.VMEM((_KMAX, _C), jnp.int32),
        pltpu.VMEM((_KMAX, _C), jnp.int32),
        pltpu.VMEM((_NB, _C, _H), jnp.float32),
        pltpu.VMEM((_RT, _H), jnp.float32),
        pltpu.VMEM_SHARED((_NPAD, _H), jnp.float32),
        [pltpu.SemaphoreType.DMA] * _NB,
        [pltpu.SemaphoreType.DMA] * _NB,
    ],
    compiler_params=pltpu.CompilerParams(use_tc_tiling_on_sc=False, disable_bounds_checks=True),
)


def _scale_body(x_ref, w_ref, degp_ref, y_ref, inv_ref):
    deg_row = 1.0 + degp_ref[0:1, :_N] + degp_ref[1:2, :_N]
    inv_col = jnp.transpose(lax.rsqrt(deg_row))
    inv = inv_col * jnp.ones((1, _H), jnp.float32)
    inv_ref[...] = inv
    xw = jnp.dot(x_ref[...], w_ref[...], preferred_element_type=jnp.float32)
    y_ref[...] = xw * inv


def _mid_body(p_ref, y_ref, inv_ref, b_ref, w_ref, y2_ref):
    ssum = p_ref[0, :_N, :] + p_ref[1, :_N, :] + y_ref[...]
    inv = inv_ref[...]
    h = jnp.maximum(ssum * inv + b_ref[...], 0.0)
    y2_ref[...] = jnp.dot(h, w_ref[...],
                          preferred_element_type=jnp.float32) * inv


def _fin_body(p_ref, y2_ref, inv_ref, b_ref, wfc_ref, bfc_ref, o_ref):
    ssum = p_ref[0, :_N, :] + p_ref[1, :_N, :] + y2_ref[...]
    h = jnp.maximum(ssum * inv_ref[...] + b_ref[...], 0.0)
    logits = jnp.dot(h, wfc_ref[...],
                     preferred_element_type=jnp.float32) + bfc_ref[...]
    m = jnp.max(logits, axis=1, keepdims=True)
    lse = m + jnp.log(jnp.sum(jnp.exp(logits - m), axis=1, keepdims=True))
    o_ref[...] = logits - lse


def _tc_call(body, *out_shapes):
    return pl.pallas_call(
        body,
        out_shape=(tuple(out_shapes) if len(out_shapes) > 1 else out_shapes[0]),
    )


def kernel(x, edge_index, W1, b1, W2, b2, Wfc, bfc):
    srcr = edge_index[0].reshape(_NCH, _C)
    dstr = edge_index[1].reshape(_NCH, _C)

    degp = _deg_call(dstr)
    y1, inv = _tc_call(
        _scale_body,
        jax.ShapeDtypeStruct((_N, _H), jnp.float32),
        jax.ShapeDtypeStruct((_N, _H), jnp.float32),
    )(x, W1, degp)

    p1 = _msg_call(y1, srcr, dstr)
    y2 = _tc_call(_mid_body, jax.ShapeDtypeStruct((_N, _H), jnp.float32))(
        p1, y1, inv, b1.reshape(1, _H), W2)

    p2 = _msg_call(y2, srcr, dstr)
    return _tc_call(_fin_body, jax.ShapeDtypeStruct((_N, _O), jnp.float32))(
        p2, y2, inv, b2.reshape(1, _H), Wfc, bfc.reshape(1, _O))


# speedup vs baseline: 61.0594x; 1.0739x over previous
import functools

import jax
import jax.numpy as jnp
from jax import lax
from jax.experimental import pallas as pl
from jax.experimental.pallas import tpu as pltpu
from jax.experimental.pallas import tpu_sc as plsc

_N = 10000
_E = 320000
_D = 128
_H = 16
_O = 2

_NC = 2
_NS = 16
_NW = _NC * _NS
_C = 128
_K0 = 88
_K1 = 68
_KMAX = _K0 + 4
_NCH = _E // _C
_RT = 640
_NPAD = _NS * _RT
_NB = 4

_sc_mesh = plsc.VectorSubcoreMesh(core_axis_name="c", subcore_axis_name="s")


def _chunk_range(c, s):
    nk = jnp.where(c == 0, _K0 + 4 * (s == 0).astype(jnp.int32), _K1)
    base0 = _K0 * s + 4 * (s > 0).astype(jnp.int32)
    base1 = _NS * _K0 + 4 + _K1 * s
    return nk, jnp.where(c == 0, base0, base1)


def _stage_idx(c, idx_hbm, base, idx_v):
    @pl.when(c == 0)
    def _():
        pltpu.sync_copy(idx_hbm.at[pl.ds(base, _KMAX)], idx_v)

    @pl.when(c != 0)
    def _():
        pltpu.sync_copy(idx_hbm.at[pl.ds(base, _K1)], idx_v.at[pl.ds(0, _K1)])


def _deg_body(ei_hbm, out_hbm, dst_v, ones_v, stage_v, deg_sh, gsem):
    c = lax.axis_index("c")
    s = lax.axis_index("s")
    nk, base = _chunk_range(c, s)

    for i in range(8):
        ones_v[pl.ds(16 * i, 16)] = jnp.ones((16,), jnp.float32)

    def _zero(i, carry):
        stage_v[pl.ds(i * 16, 16)] = jnp.zeros((16,), jnp.float32)
        return carry

    lax.fori_loop(0, _RT // 16, _zero, 0)

    _stage_idx(c, ei_hbm.at[1], base, dst_v)
    pltpu.sync_copy(stage_v, deg_sh.at[pl.ds(s * _RT, _RT)])
    plsc.subcore_barrier()

    def _group(g, carry):
        cps = [pltpu.async_copy(ones_v, deg_sh.at[dst_v.at[g * 4 + b]], gsem,
                                add=True)
               for b in range(4)]
        for cp in cps:
            cp.wait()
        return carry

    lax.fori_loop(0, nk // 4, _group, 0)
    plsc.subcore_barrier()
    pltpu.sync_copy(deg_sh.at[pl.ds(s * _RT, _RT)], stage_v)
    pltpu.sync_copy(stage_v, out_hbm.at[c, pl.ds(s * _RT, _RT)])


_deg_call = pl.kernel(
    _deg_body,
    out_type=jax.ShapeDtypeStruct((_NC, _NPAD), jnp.float32),
    mesh=_sc_mesh,
    scratch_types=[
        pltpu.VMEM((_KMAX, _C), jnp.int32),
        pltpu.VMEM((_C,), jnp.float32),
        pltpu.VMEM((_RT,), jnp.float32),
        pltpu.VMEM_SHARED((_NPAD,), jnp.float32),
        pltpu.SemaphoreType.DMA,
    ],
    compiler_params=pltpu.CompilerParams(use_tc_tiling_on_sc=False, disable_bounds_checks=True),
)


def _msg_body(y_hbm, ei_hbm, out_hbm, src_v, dst_v, rows_v, stage_v,
              acc_sh, gsems, ssems):
    c = lax.axis_index("c")
    s = lax.axis_index("s")
    nk, base = _chunk_range(c, s)

    def _zero(i, carry):
        stage_v[i] = jnp.zeros((16,), jnp.float32)
        return carry

    lax.fori_loop(0, _RT, _zero, 0)

    _stage_idx(c, ei_hbm.at[0], base, src_v)
    _stage_idx(c, ei_hbm.at[1], base, dst_v)
    pltpu.sync_copy(stage_v, acc_sh.at[pl.ds(s * _RT, _RT)])
    plsc.subcore_barrier()

    def _gather(j, b):
        return pltpu.async_copy(y_hbm.at[src_v.at[j]], rows_v.at[b], gsems[b])

    def _scatter(j, b):
        return pltpu.async_copy(rows_v.at[b], acc_sh.at[dst_v.at[j]], ssems[b],
                                add=True)

    for b in range(_NB):
        _gather(b, b)

    def _quad(q, carry):
        j0 = q * _NB
        for b in range(_NB):
            pltpu.make_async_copy(y_hbm.at[src_v.at[j0 + b]], rows_v.at[b],
                                  gsems[b]).wait()
            _scatter(j0 + b, b)
        for b in range(_NB):
            pltpu.make_async_copy(rows_v.at[b], acc_sh.at[dst_v.at[j0 + b]],
                                  ssems[b]).wait()
            _gather(j0 + _NB + b, b)
        return carry

    lax.fori_loop(0, nk // _NB - 1, _quad, 0)

    jlast = nk - _NB
    for b in range(_NB):
        pltpu.make_async_copy(y_hbm.at[src_v.at[jlast + b]], rows_v.at[b],
                              gsems[b]).wait()
        _scatter(jlast + b, b)
    for b in range(_NB):
        pltpu.make_async_copy(rows_v.at[b], acc_sh.at[dst_v.at[jlast + b]],
                              ssems[b]).wait()

    plsc.subcore_barrier()
    pltpu.sync_copy(acc_sh.at[pl.ds(s * _RT, _RT)], stage_v)
    pltpu.sync_copy(stage_v, out_hbm.at[c, pl.ds(s * _RT, _RT)])


_msg_call = pl.kernel(
    _msg_body,
    out_type=jax.ShapeDtypeStruct((_NC, _NPAD, _H), jnp.float32),
    mesh=_sc_mesh,
    scratch_types=[
        pltpu.VMEM((_KMAX, _C), jnp.int32),
        pltpu.VMEM((_KMAX, _C), jnp.int32),
        pltpu.VMEM((_NB, _C, _H), jnp.float32),
        pltpu.VMEM((_RT, _H), jnp.float32),
        pltpu.VMEM_SHARED((_NPAD, _H), jnp.float32),
        [pltpu.SemaphoreType.DMA] * _NB,
        [pltpu.SemaphoreType.DMA] * _NB,
    ],
    compiler_params=pltpu.CompilerParams(use_tc_tiling_on_sc=False, disable_bounds_checks=True),
)


def _scale_body(x_ref, w_ref, degp_ref, y_ref, inv_ref):
    deg_row = 1.0 + degp_ref[0:1, :_N] + degp_ref[1:2, :_N]
    inv_col = jnp.transpose(lax.rsqrt(deg_row))
    inv = inv_col * jnp.ones((1, _H), jnp.float32)
    inv_ref[...] = inv
    xw = jnp.dot(x_ref[...], w_ref[...], preferred_element_type=jnp.float32)
    y_ref[...] = xw * inv


def _sum_p(p_ref):
    return p_ref[0, :_N, :] + p_ref[1, :_N, :]


def _mid_body(p_ref, y_ref, inv_ref, b_ref, w_ref, y2_ref):
    ssum = _sum_p(p_ref) + y_ref[...]
    inv = inv_ref[...]
    h = jnp.maximum(ssum * inv + b_ref[...], 0.0)
    y2_ref[...] = jnp.dot(h, w_ref[...],
                          preferred_element_type=jnp.float32) * inv


def _fin_body(p_ref, y2_ref, inv_ref, b_ref, wfc_ref, bfc_ref, o_ref):
    ssum = _sum_p(p_ref) + y2_ref[...]
    h = jnp.maximum(ssum * inv_ref[...] + b_ref[...], 0.0)
    logits = jnp.dot(h, wfc_ref[...],
                     preferred_element_type=jnp.float32) + bfc_ref[...]
    m = jnp.max(logits, axis=1, keepdims=True)
    lse = m + jnp.log(jnp.sum(jnp.exp(logits - m), axis=1, keepdims=True))
    o_ref[...] = logits - lse


def _tc_call(body, *out_shapes):
    return pl.pallas_call(
        body,
        out_shape=(tuple(out_shapes) if len(out_shapes) > 1 else out_shapes[0]),
    )


def kernel(x, edge_index, W1, b1, W2, b2, Wfc, bfc):
    ei3 = edge_index.reshape(_NC, _NCH, _C)

    degp = _deg_call(ei3)
    y1, inv = _tc_call(
        _scale_body,
        jax.ShapeDtypeStruct((_N, _H), jnp.float32),
        jax.ShapeDtypeStruct((_N, _H), jnp.float32),
    )(x, W1, degp)

    p1 = _msg_call(y1, ei3)
    y2 = _tc_call(_mid_body, jax.ShapeDtypeStruct((_N, _H), jnp.float32))(
        p1, y1, inv, b1.reshape(1, _H), W2)

    p2 = _msg_call(y2, ei3)
    return _tc_call(_fin_body, jax.ShapeDtypeStruct((_N, _O), jnp.float32))(
        p2, y2, inv, b2.reshape(1, _H), Wfc, bfc.reshape(1, _O))


# speedup vs baseline: 61.2010x; 1.0023x over previous
import functools

import jax
import jax.numpy as jnp
from jax import lax
from jax.experimental import pallas as pl
from jax.experimental.pallas import tpu as pltpu
from jax.experimental.pallas import tpu_sc as plsc

_N = 10000
_E = 320000
_D = 128
_H = 16
_O = 2

_NC = 2
_NS = 16
_NW = _NC * _NS
_C = 128
_K0 = 88
_K1 = 68
_KMAX = _K0 + 4
_NCH = _E // _C
_RT = 640
_NPAD = _NS * _RT
_NB = 4

_sc_mesh = plsc.VectorSubcoreMesh(core_axis_name="c", subcore_axis_name="s")


def _chunk_range(c, s):
    nk = jnp.where(c == 0, _K0 + 4 * (s == 0).astype(jnp.int32), _K1)
    base0 = _K0 * s + 4 * (s > 0).astype(jnp.int32)
    base1 = _NS * _K0 + 4 + _K1 * s
    return nk, jnp.where(c == 0, base0, base1)


def _stage_idx(c, idx_hbm, base, idx_v):
    @pl.when(c == 0)
    def _():
        pltpu.sync_copy(idx_hbm.at[pl.ds(base, _KMAX)], idx_v)

    @pl.when(c != 0)
    def _():
        pltpu.sync_copy(idx_hbm.at[pl.ds(base, _K1)], idx_v.at[pl.ds(0, _K1)])


def _deg_body(ei_hbm, out_hbm, dst_v, ones_v, stage_v, deg_sh, gsem):
    c = lax.axis_index("c")
    s = lax.axis_index("s")
    nk, base = _chunk_range(c, s)

    for i in range(8):
        ones_v[pl.ds(16 * i, 16)] = jnp.ones((16,), jnp.float32)

    def _zero(i, carry):
        stage_v[pl.ds(i * 16, 16)] = jnp.zeros((16,), jnp.float32)
        return carry

    lax.fori_loop(0, _RT // 16, _zero, 0)

    _stage_idx(c, ei_hbm.at[1], base, dst_v)
    pltpu.sync_copy(stage_v, deg_sh.at[pl.ds(s * _RT, _RT)])
    plsc.subcore_barrier()

    def _fire(j0):
        for b in range(4):
            pltpu.async_copy(ones_v, deg_sh.at[dst_v.at[j0 + b]], gsem,
                             add=True)

    def _drain(j0):
        for b in range(4):
            pltpu.make_async_copy(ones_v, deg_sh.at[dst_v.at[j0 + b]],
                                  gsem).wait()

    _fire(0)

    def _group(g, carry):
        _fire(g * 4 + 4)
        _drain(g * 4)
        return carry

    lax.fori_loop(0, nk // 4 - 1, _group, 0)
    _drain(nk - 4)
    plsc.subcore_barrier()
    pltpu.sync_copy(deg_sh.at[pl.ds(s * _RT, _RT)], stage_v)
    pltpu.sync_copy(stage_v, out_hbm.at[c, pl.ds(s * _RT, _RT)])


_deg_call = pl.kernel(
    _deg_body,
    out_type=jax.ShapeDtypeStruct((_NC, _NPAD), jnp.float32),
    mesh=_sc_mesh,
    scratch_types=[
        pltpu.VMEM((_KMAX, _C), jnp.int32),
        pltpu.VMEM((_C,), jnp.float32),
        pltpu.VMEM((_RT,), jnp.float32),
        pltpu.VMEM_SHARED((_NPAD,), jnp.float32),
        pltpu.SemaphoreType.DMA,
    ],
    compiler_params=pltpu.CompilerParams(use_tc_tiling_on_sc=False, disable_bounds_checks=True),
)


def _msg_body(y_hbm, ei_hbm, out_hbm, src_v, dst_v, rows_v, stage_v,
              acc_sh, gsems, ssems):
    c = lax.axis_index("c")
    s = lax.axis_index("s")
    nk, base = _chunk_range(c, s)

    def _zero(i, carry):
        stage_v[i] = jnp.zeros((16,), jnp.float32)
        return carry

    lax.fori_loop(0, _RT, _zero, 0)

    _stage_idx(c, ei_hbm.at[0], base, src_v)
    _stage_idx(c, ei_hbm.at[1], base, dst_v)
    pltpu.sync_copy(stage_v, acc_sh.at[pl.ds(s * _RT, _RT)])
    plsc.subcore_barrier()

    def _gather(j, b):
        return pltpu.async_copy(y_hbm.at[src_v.at[j]], rows_v.at[b], gsems[b])

    def _scatter(j, b):
        return pltpu.async_copy(rows_v.at[b], acc_sh.at[dst_v.at[j]], ssems[b],
                                add=True)

    for b in range(_NB):
        _gather(b, b)

    def _quad(q, carry):
        j0 = q * _NB
        for b in range(_NB):
            pltpu.make_async_copy(y_hbm.at[src_v.at[j0 + b]], rows_v.at[b],
                                  gsems[b]).wait()
            _scatter(j0 + b, b)
        for b in range(_NB):
            pltpu.make_async_copy(rows_v.at[b], acc_sh.at[dst_v.at[j0 + b]],
                                  ssems[b]).wait()
            _gather(j0 + _NB + b, b)
        return carry

    lax.fori_loop(0, nk // _NB - 1, _quad, 0)

    jlast = nk - _NB
    for b in range(_NB):
        pltpu.make_async_copy(y_hbm.at[src_v.at[jlast + b]], rows_v.at[b],
                              gsems[b]).wait()
        _scatter(jlast + b, b)
    for b in range(_NB):
        pltpu.make_async_copy(rows_v.at[b], acc_sh.at[dst_v.at[jlast + b]],
                              ssems[b]).wait()

    plsc.subcore_barrier()
    pltpu.sync_copy(acc_sh.at[pl.ds(s * _RT, _RT)], stage_v)
    pltpu.sync_copy(stage_v, out_hbm.at[c, pl.ds(s * _RT, _RT)])


_msg_call = pl.kernel(
    _msg_body,
    out_type=jax.ShapeDtypeStruct((_NC, _NPAD, _H), jnp.float32),
    mesh=_sc_mesh,
    scratch_types=[
        pltpu.VMEM((_KMAX, _C), jnp.int32),
        pltpu.VMEM((_KMAX, _C), jnp.int32),
        pltpu.VMEM((_NB, _C, _H), jnp.float32),
        pltpu.VMEM((_RT, _H), jnp.float32),
        pltpu.VMEM_SHARED((_NPAD, _H), jnp.float32),
        [pltpu.SemaphoreType.DMA] * _NB,
        [pltpu.SemaphoreType.DMA] * _NB,
    ],
    compiler_params=pltpu.CompilerParams(use_tc_tiling_on_sc=False, disable_bounds_checks=True),
)


def _mm1_body(x_ref, w_ref, o_ref):
    o_ref[...] = jnp.dot(x_ref[...], w_ref[...],
                         preferred_element_type=jnp.float32)


def _scale_body(xw_ref, degp_ref, y_ref, inv_ref):
    deg_row = 1.0 + degp_ref[0:1, :_N] + degp_ref[1:2, :_N]
    inv_col = jnp.transpose(lax.rsqrt(deg_row))
    inv = inv_col * jnp.ones((1, _H), jnp.float32)
    inv_ref[...] = inv
    y_ref[...] = xw_ref[...] * inv


def _sum_p(p_ref):
    return p_ref[0, :_N, :] + p_ref[1, :_N, :]


def _mid_body(p_ref, y_ref, inv_ref, b_ref, w_ref, y2_ref):
    ssum = _sum_p(p_ref) + y_ref[...]
    inv = inv_ref[...]
    h = jnp.maximum(ssum * inv + b_ref[...], 0.0)
    y2_ref[...] = jnp.dot(h, w_ref[...],
                          preferred_element_type=jnp.float32) * inv


def _fin_body(p_ref, y2_ref, inv_ref, b_ref, wfc_ref, bfc_ref, o_ref):
    ssum = _sum_p(p_ref) + y2_ref[...]
    h = jnp.maximum(ssum * inv_ref[...] + b_ref[...], 0.0)
    logits = jnp.dot(h, wfc_ref[...],
                     preferred_element_type=jnp.float32) + bfc_ref[...]
    m = jnp.max(logits, axis=1, keepdims=True)
    lse = m + jnp.log(jnp.sum(jnp.exp(logits - m), axis=1, keepdims=True))
    o_ref[...] = logits - lse


def _tc_call(body, *out_shapes):
    return pl.pallas_call(
        body,
        out_shape=(tuple(out_shapes) if len(out_shapes) > 1 else out_shapes[0]),
    )


def kernel(x, edge_index, W1, b1, W2, b2, Wfc, bfc):
    ei3 = edge_index.reshape(_NC, _NCH, _C)

    xw1 = _tc_call(_mm1_body, jax.ShapeDtypeStruct((_N, _H), jnp.float32))(x, W1)
    degp = _deg_call(ei3)
    y1, inv = _tc_call(
        _scale_body,
        jax.ShapeDtypeStruct((_N, _H), jnp.float32),
        jax.ShapeDtypeStruct((_N, _H), jnp.float32),
    )(xw1, degp)

    p1 = _msg_call(y1, ei3)
    y2 = _tc_call(_mid_body, jax.ShapeDtypeStruct((_N, _H), jnp.float32))(
        p1, y1, inv, b1.reshape(1, _H), W2)

    p2 = _msg_call(y2, ei3)
    return _tc_call(_fin_body, jax.ShapeDtypeStruct((_N, _O), jnp.float32))(
        p2, y2, inv, b2.reshape(1, _H), Wfc, bfc.reshape(1, _O))
